# baseline (speedup 1.0000x reference)
import functools

import jax
import jax.numpy as jnp
from jax import lax
from jax.experimental import pallas as pl
from jax.experimental.pallas import tpu as pltpu
from jax.experimental.pallas import tpu_sc as plsc

N = 10000
E = 320000
F = 128
HD = 128
G = 64
HL = 3 * HD

REL_SRC = (0, 1, 0, 1, 2, 0, 1)
REL_DST = (1, 0, 2, 2, 2, 0, 1)
DST_RELS = ((1, 5), (0, 6), (2, 3, 4))

SLOT_OF_REL = (0, 4, 1, 5, 8, 2, 6)
ROOT_SLOT = (3, 7, 9)
DST_PAIRS = tuple(tuple((SLOT_OF_REL[r], r) for r in rels)
                  for rels in DST_RELS)

NCORE = 2
NSUB = 16
NW = NCORE * NSUB
EW = E // NW
CH = 125
NHALF = 4
NH = EW // (CH * NHALF)
RPS = 624
TAIL = N - NSUB * RPS

BLK = 1000
NB = N // BLK


def _xform_body(x_ref, w_ref, b_ref, o_ref):
    o_ref[...] = (jnp.dot(x_ref[0], w_ref[0], preferred_element_type=jnp.float32)
                  + b_ref[0])[None]


def _xform(x3, ws, bs):
    return pl.pallas_call(
        _xform_body,
        grid=(NB, 10),
        in_specs=[
            pl.BlockSpec((1, BLK, F), lambda i, o: (o // 4, i, 0)),
            pl.BlockSpec((1, F, HD), lambda i, o: (o, 0, 0)),
            pl.BlockSpec((1, 1, HD), lambda i, o: (o, 0, 0)),
        ],
        out_specs=pl.BlockSpec((1, BLK, HD), lambda i, o: (o, i, 0)),
        out_shape=jax.ShapeDtypeStruct((10, N, HD), jnp.float32),
    )(x3, ws, bs)


def _combine_body(a_ref, r_ref, o_ref):
    o_ref[...] = jnp.maximum(a_ref[0, 0] + a_ref[0, 1] + r_ref[0], 0.0)[None]


def _combine(acc, t_full):
    return pl.pallas_call(
        _combine_body,
        grid=(3, NB),
        in_specs=[
            pl.BlockSpec((1, NCORE, BLK, HD), lambda t, i: (t, 0, i, 0)),
            pl.BlockSpec((1, BLK, HD),
                         lambda t, i: (jnp.minimum(4 * t + 3, 9), i, 0)),
        ],
        out_specs=pl.BlockSpec((1, BLK, HD), lambda t, i: (t, i, 0)),
        out_shape=jax.ShapeDtypeStruct((3, N, HD), jnp.float32),
    )(acc, t_full)


def _pool_body(b_ref, a_ref, r_ref, ps_ref, cs_ref):
    h = jnp.maximum(a_ref[0, 0] + a_ref[0, 1] + r_ref[0], 0.0)
    seg = lax.broadcasted_iota(jnp.int32, (G, N), 0)
    onehot = (jnp.broadcast_to(b_ref[0], (G, N)) == seg).astype(jnp.float32)
    ps_ref[0] = jnp.dot(onehot, h, preferred_element_type=jnp.float32)
    cs_ref[0] = jnp.broadcast_to(jnp.sum(onehot, axis=1, keepdims=True), (G, HD))


def _pool(batch3, acc, t_full):
    return pl.pallas_call(
        _pool_body,
        grid=(3,),
        in_specs=[
            pl.BlockSpec((1, 1, N), lambda t: (t, 0, 0)),
            pl.BlockSpec((1, NCORE, N, HD), lambda t: (t, 0, 0, 0)),
            pl.BlockSpec((1, N, HD), lambda t: (jnp.minimum(4 * t + 3, 9), 0, 0)),
        ],
        out_specs=[
            pl.BlockSpec((1, G, HD), lambda t: (t, 0, 0)),
            pl.BlockSpec((1, G, HD), lambda t: (t, 0, 0)),
        ],
        out_shape=[
            jax.ShapeDtypeStruct((3, G, HD), jnp.float32),
            jax.ShapeDtypeStruct((3, G, HD), jnp.float32),
        ],
    )(batch3, acc, t_full)


def _mlp_body(ps_ref, cs_ref, w1_ref, b1_ref, w2_ref, b2_ref, w3_ref, b3_ref,
              wo_ref, bo_ref, o_ref):
    pool = ps_ref[...] / jnp.maximum(cs_ref[...], 1.0)
    h = jnp.concatenate([pool[0], pool[1], pool[2]], axis=1)
    h = jnp.maximum(jnp.dot(h, w1_ref[...], preferred_element_type=jnp.float32)
                    + b1_ref[...], 0.0)
    h = jnp.maximum(jnp.dot(h, w2_ref[...], preferred_element_type=jnp.float32)
                    + b2_ref[...], 0.0)
    h = jnp.maximum(jnp.dot(h, w3_ref[...], preferred_element_type=jnp.float32)
                    + b3_ref[...], 0.0)
    o_ref[...] = jnp.dot(h, wo_ref[...], preferred_element_type=jnp.float32) + bo_ref[...]


def _mlp(ps, cs, w1, b1, w2, b2, w3, b3, wo_pad, bo_pad):
    return pl.pallas_call(
        _mlp_body,
        out_shape=jax.ShapeDtypeStruct((G, HD), jnp.float32),
    )(ps, cs, w1, b1, w2, b2, w3, b3, wo_pad, bo_pad)


def _sc_scatter_body(y_hbm, src_hbm, dst_hbm, z_hbm, out_hbm,
                     src_v, dst_v, rows0, rows1, acc, sem0, sem1):
    c = lax.axis_index("c")
    s = lax.axis_index("s")
    wid = s * NCORE + c
    row0 = s * RPS
    for d in range(3):
        pltpu.sync_copy(z_hbm.at[pl.ds(0, RPS)], acc.at[pl.ds(row0, RPS)])

        @pl.when(s == NSUB - 1)
        def _():
            pltpu.sync_copy(z_hbm.at[pl.ds(0, TAIL)],
                            acc.at[pl.ds(N - TAIL, TAIL)])

        plsc.subcore_barrier()
        for slot, r in DST_PAIRS[d]:
            y_slot = y_hbm.at[slot]
            for h in range(NHALF):
                pltpu.sync_copy(src_hbm.at[r, wid, h], src_v)
                pltpu.sync_copy(dst_hbm.at[r, wid, h], dst_v)

                pltpu.async_copy(y_slot.at[src_v.at[0]], rows0, sem0)

                def body(p, carry, y_slot=y_slot):
                    c0, c1, c2 = 2 * p, 2 * p + 1, 2 * p + 2
                    pltpu.async_copy(y_slot.at[src_v.at[c1]], rows1, sem1)
                    pltpu.make_async_copy(y_slot.at[src_v.at[c0]], rows0, sem0).wait()
                    pltpu.sync_copy(rows0, acc.at[dst_v.at[c0]], add=True)
                    pltpu.async_copy(y_slot.at[src_v.at[c2]], rows0, sem0)
                    pltpu.make_async_copy(y_slot.at[src_v.at[c1]], rows1, sem1).wait()
                    pltpu.sync_copy(rows1, acc.at[dst_v.at[c1]], add=True)
                    return carry

                lax.fori_loop(0, NH // 2 - 1, body, 0)
                pltpu.async_copy(y_slot.at[src_v.at[NH - 1]], rows1, sem1)
                pltpu.make_async_copy(y_slot.at[src_v.at[NH - 2]], rows0, sem0).wait()
                pltpu.sync_copy(rows0, acc.at[dst_v.at[NH - 2]], add=True)
                pltpu.make_async_copy(y_slot.at[src_v.at[NH - 1]], rows1, sem1).wait()
                pltpu.sync_copy(rows1, acc.at[dst_v.at[NH - 1]], add=True)
        plsc.subcore_barrier()
        pltpu.sync_copy(acc.at[pl.ds(row0, RPS)],
                        out_hbm.at[d, c, pl.ds(row0, RPS)])

        @pl.when(s == NSUB - 1)
        def _():
            pltpu.sync_copy(acc.at[pl.ds(N - TAIL, TAIL)],
                            out_hbm.at[d, c, pl.ds(N - TAIL, TAIL)])

        plsc.subcore_barrier()


@functools.cache
def _sc_scatter_kernel():
    return pl.kernel(
        _sc_scatter_body,
        out_type=jax.ShapeDtypeStruct((3, NCORE, N, HD), jnp.float32),
        mesh=plsc.VectorSubcoreMesh(core_axis_name="c", subcore_axis_name="s",
                                    num_cores=NCORE, num_subcores=NSUB),
        scratch_types=[
            pltpu.VMEM((NH, CH), jnp.int32),
            pltpu.VMEM((NH, CH), jnp.int32),
            pltpu.VMEM((CH, HD), jnp.float32),
            pltpu.VMEM((CH, HD), jnp.float32),
            pltpu.VMEM_SHARED((N, HD), jnp.float32),
            pltpu.SemaphoreType.DMA,
            pltpu.SemaphoreType.DMA,
        ],
    )


def _sc_scatter(y, src_idx, dst_idx, zeros):
    return _sc_scatter_kernel()(y, src_idx, dst_idx, zeros)


def kernel(x_a0, x_a1, x_b, ei_a0_a1, ei_a1_a0, ei_a0_b, ei_a1_b, ei_b_b,
           ei_a0_a0, ei_a1_a1, batch_a0, batch_a1, batch_b, W_rel, b_rel,
           W_root, W1, b1, W2, b2, W3, b3, Wout, bout):
    eis = (ei_a0_a1, ei_a1_a0, ei_a0_b, ei_a1_b, ei_b_b, ei_a0_a0, ei_a1_a1)

    src_idx = jnp.stack([e[0].reshape(NW, NHALF, NH, CH) for e in eis])
    dst_idx = jnp.stack([e[1].reshape(NW, NHALF, NH, CH) for e in eis])
    zeros = jnp.zeros((RPS, HD), jnp.float32)

    def layer_weights(l):
        zb = jnp.zeros((1, HD), jnp.float32)
        w_slots, b_slots = [None] * 10, [None] * 10
        for r in range(7):
            w_slots[SLOT_OF_REL[r]] = W_rel[l, r]
            b_slots[SLOT_OF_REL[r]] = zb
        for d in range(3):
            w_slots[ROOT_SLOT[d]] = sum(W_root[l, r] for r in DST_RELS[d])
            b_slots[ROOT_SLOT[d]] = sum(b_rel[l, r] for r in DST_RELS[d])[None, :]
        return jnp.stack(w_slots), jnp.stack(b_slots)

    x3 = jnp.stack([x_a0, x_a1, x_b])
    ws0, bs0 = layer_weights(0)
    t0 = _xform(x3, ws0, bs0)
    a0 = _sc_scatter(t0, src_idx, dst_idx, zeros)
    h1 = _combine(a0, t0)

    ws1, bs1 = layer_weights(1)
    t1 = _xform(h1, ws1, bs1)
    a1 = _sc_scatter(t1, src_idx, dst_idx, zeros)

    batch3 = jnp.stack([batch_a0, batch_a1, batch_b])[:, None, :]
    ps, cs = _pool(batch3, a1, t1)

    wo_pad = jnp.pad(Wout, ((0, 0), (0, HD - 1)))
    bo_pad = jnp.pad(bout[None, :], ((0, 0), (0, HD - 1)))
    out = _mlp(ps, cs, W1, b1[None, :], W2, b2[None, :], W3, b3[None, :],
               wo_pad, bo_pad)
    return out[:, 0]


# speedup vs baseline: 1.0188x; 1.0188x over previous
import functools

import jax
import jax.numpy as jnp
from jax import lax
from jax.experimental import pallas as pl
from jax.experimental.pallas import tpu as pltpu
from jax.experimental.pallas import tpu_sc as plsc

N = 10000
E = 320000
F = 128
HD = 128
G = 64
HL = 3 * HD

REL_SRC = (0, 1, 0, 1, 2, 0, 1)
REL_DST = (1, 0, 2, 2, 2, 0, 1)
DST_RELS = ((1, 5), (0, 6), (2, 3, 4))

SLOT_OF_REL = (0, 4, 1, 5, 8, 2, 6)
ROOT_SLOT = (3, 7, 9)
DST_PAIRS = tuple(tuple((SLOT_OF_REL[r], r) for r in rels)
                  for rels in DST_RELS)

NCORE = 2
NSUB = 16
NW = NCORE * NSUB
EW = E // NW
CH = 100
NHALF = 2
NH = EW // (CH * NHALF)
RPS = 624
TAIL = N - NSUB * RPS

BLK = 1000
NB = N // BLK


def _xform_body(x_ref, w_ref, b_ref, o_ref):
    o_ref[...] = (jnp.dot(x_ref[0], w_ref[0], preferred_element_type=jnp.float32)
                  + b_ref[0])[None]


def _xform(x3, ws, bs, src_div=4):
    nout = ws.shape[0]
    return pl.pallas_call(
        _xform_body,
        grid=(NB, nout),
        in_specs=[
            pl.BlockSpec((1, BLK, F), lambda i, o: (o // src_div, i, 0)),
            pl.BlockSpec((1, F, HD), lambda i, o: (o, 0, 0)),
            pl.BlockSpec((1, 1, HD), lambda i, o: (o, 0, 0)),
        ],
        out_specs=pl.BlockSpec((1, BLK, HD), lambda i, o: (o, i, 0)),
        out_shape=jax.ShapeDtypeStruct((nout, N, HD), jnp.float32),
    )(x3, ws, bs)


def _combine_body(a_ref, r_ref, o_ref):
    o_ref[...] = jnp.maximum(a_ref[0, 0] + a_ref[0, 1] + r_ref[0], 0.0)[None]


def _combine(acc, t_planes, root_fn):
    k = acc.shape[0]
    return pl.pallas_call(
        _combine_body,
        grid=(k, NB),
        in_specs=[
            pl.BlockSpec((1, NCORE, BLK, HD), lambda t, i: (t, 0, i, 0)),
            pl.BlockSpec((1, BLK, HD), lambda t, i: (root_fn(t), i, 0)),
        ],
        out_specs=pl.BlockSpec((1, BLK, HD), lambda t, i: (t, i, 0)),
        out_shape=jax.ShapeDtypeStruct((k, N, HD), jnp.float32),
    )(acc, t_planes)


def _pool_body(b_ref, a_ref, r_ref, ps_ref, cs_ref):
    h = jnp.maximum(a_ref[0, 0] + a_ref[0, 1] + r_ref[0], 0.0)
    seg = lax.broadcasted_iota(jnp.int32, (G, N), 0)
    onehot = (jnp.broadcast_to(b_ref[0], (G, N)) == seg).astype(jnp.float32)
    ps_ref[0] = jnp.dot(onehot, h, preferred_element_type=jnp.float32)
    cs_ref[0] = jnp.broadcast_to(jnp.sum(onehot, axis=1, keepdims=True), (G, HD))


def _pool(batchk, acc, t_planes, root_fn):
    k = acc.shape[0]
    return pl.pallas_call(
        _pool_body,
        grid=(k,),
        in_specs=[
            pl.BlockSpec((1, 1, N), lambda t: (t, 0, 0)),
            pl.BlockSpec((1, NCORE, N, HD), lambda t: (t, 0, 0, 0)),
            pl.BlockSpec((1, N, HD), lambda t: (root_fn(t), 0, 0)),
        ],
        out_specs=[
            pl.BlockSpec((1, G, HD), lambda t: (t, 0, 0)),
            pl.BlockSpec((1, G, HD), lambda t: (t, 0, 0)),
        ],
        out_shape=[
            jax.ShapeDtypeStruct((k, G, HD), jnp.float32),
            jax.ShapeDtypeStruct((k, G, HD), jnp.float32),
        ],
    )(batchk, acc, t_planes)


def _mlp_body(ps_ref, cs_ref, w1_ref, b1_ref, w2_ref, b2_ref, w3_ref, b3_ref,
              wo_ref, bo_ref, o_ref):
    pool = ps_ref[...] / jnp.maximum(cs_ref[...], 1.0)
    h = jnp.concatenate([pool[0], pool[1], pool[2]], axis=1)
    h = jnp.maximum(jnp.dot(h, w1_ref[...], preferred_element_type=jnp.float32)
                    + b1_ref[...], 0.0)
    h = jnp.maximum(jnp.dot(h, w2_ref[...], preferred_element_type=jnp.float32)
                    + b2_ref[...], 0.0)
    h = jnp.maximum(jnp.dot(h, w3_ref[...], preferred_element_type=jnp.float32)
                    + b3_ref[...], 0.0)
    o_ref[...] = jnp.dot(h, wo_ref[...], preferred_element_type=jnp.float32) + bo_ref[...]


def _mlp(ps, cs, w1, b1, w2, b2, w3, b3, wo_pad, bo_pad):
    return pl.pallas_call(
        _mlp_body,
        out_shape=jax.ShapeDtypeStruct((G, HD), jnp.float32),
    )(ps, cs, w1, b1, w2, b2, w3, b3, wo_pad, bo_pad)


def _sc_scatter_body(groups, slot_map, n_y, *refs):
    ys = refs[:n_y]
    (src_hbm, dst_hbm, z_hbm, out_hbm,
     src_v, dst_v, rows0, rows1, acc, sem0, sem1) = refs[n_y:]
    c = lax.axis_index("c")
    s = lax.axis_index("s")
    wid = s * NCORE + c
    row0 = s * RPS
    for gi, d in enumerate(groups):
        pltpu.sync_copy(z_hbm.at[pl.ds(0, RPS)], acc.at[pl.ds(row0, RPS)])

        @pl.when(s == NSUB - 1)
        def _():
            pltpu.sync_copy(z_hbm.at[pl.ds(0, TAIL)],
                            acc.at[pl.ds(N - TAIL, TAIL)])

        plsc.subcore_barrier()
        for slot, r in DST_PAIRS[d]:
            ai, plane = slot_map[slot]
            y_slot = ys[ai].at[plane]
            for h in range(NHALF):
                pltpu.sync_copy(src_hbm.at[r, wid, h], src_v)
                pltpu.sync_copy(dst_hbm.at[r, wid, h], dst_v)

                pltpu.async_copy(y_slot.at[src_v.at[0]], rows0, sem0)

                def body(p, carry, y_slot=y_slot):
                    c0, c1, c2 = 2 * p, 2 * p + 1, 2 * p + 2
                    pltpu.async_copy(y_slot.at[src_v.at[c1]], rows1, sem1)
                    pltpu.make_async_copy(y_slot.at[src_v.at[c0]], rows0, sem0).wait()
                    pltpu.sync_copy(rows0, acc.at[dst_v.at[c0]], add=True)
                    pltpu.async_copy(y_slot.at[src_v.at[c2]], rows0, sem0)
                    pltpu.make_async_copy(y_slot.at[src_v.at[c1]], rows1, sem1).wait()
                    pltpu.sync_copy(rows1, acc.at[dst_v.at[c1]], add=True)
                    return carry

                lax.fori_loop(0, NH // 2 - 1, body, 0)
                pltpu.async_copy(y_slot.at[src_v.at[NH - 1]], rows1, sem1)
                pltpu.make_async_copy(y_slot.at[src_v.at[NH - 2]], rows0, sem0).wait()
                pltpu.sync_copy(rows0, acc.at[dst_v.at[NH - 2]], add=True)
                pltpu.make_async_copy(y_slot.at[src_v.at[NH - 1]], rows1, sem1).wait()
                pltpu.sync_copy(rows1, acc.at[dst_v.at[NH - 1]], add=True)
        plsc.subcore_barrier()
        pltpu.sync_copy(acc.at[pl.ds(row0, RPS)],
                        out_hbm.at[gi, c, pl.ds(row0, RPS)])

        @pl.when(s == NSUB - 1)
        def _():
            pltpu.sync_copy(acc.at[pl.ds(N - TAIL, TAIL)],
                            out_hbm.at[gi, c, pl.ds(N - TAIL, TAIL)])

        plsc.subcore_barrier()


@functools.cache
def _sc_scatter_kernel(groups, slot_map, n_y):
    return pl.kernel(
        functools.partial(_sc_scatter_body, groups, dict(slot_map), n_y),
        out_type=jax.ShapeDtypeStruct((len(groups), NCORE, N, HD), jnp.float32),
        mesh=plsc.VectorSubcoreMesh(core_axis_name="c", subcore_axis_name="s",
                                    num_cores=NCORE, num_subcores=NSUB),
        scratch_types=[
            pltpu.VMEM((NH, CH), jnp.int32),
            pltpu.VMEM((NH, CH), jnp.int32),
            pltpu.VMEM((CH, HD), jnp.float32),
            pltpu.VMEM((CH, HD), jnp.float32),
            pltpu.VMEM_SHARED((N, HD), jnp.float32),
            pltpu.SemaphoreType.DMA,
            pltpu.SemaphoreType.DMA,
        ],
    )


def _sc_scatter(groups, ys, src_idx, dst_idx, zeros, slot_map=None):
    if slot_map is None:
        slot_map = {slot: (0, slot) for d in groups for slot, _ in DST_PAIRS[d]}
    k = _sc_scatter_kernel(tuple(groups), tuple(sorted(slot_map.items())), len(ys))
    return k(*ys, src_idx, dst_idx, zeros)


def kernel(x_a0, x_a1, x_b, ei_a0_a1, ei_a1_a0, ei_a0_b, ei_a1_b, ei_b_b,
           ei_a0_a0, ei_a1_a1, batch_a0, batch_a1, batch_b, W_rel, b_rel,
           W_root, W1, b1, W2, b2, W3, b3, Wout, bout):
    eis = (ei_a0_a1, ei_a1_a0, ei_a0_b, ei_a1_b, ei_b_b, ei_a0_a0, ei_a1_a1)

    src_idx = jnp.stack([e[0].reshape(NW, NHALF, NH, CH) for e in eis])
    dst_idx = jnp.stack([e[1].reshape(NW, NHALF, NH, CH) for e in eis])
    zeros = jnp.zeros((RPS, HD), jnp.float32)

    def layer_weights(l):
        zb = jnp.zeros((1, HD), jnp.float32)
        w_slots, b_slots = [None] * 10, [None] * 10
        for r in range(7):
            w_slots[SLOT_OF_REL[r]] = W_rel[l, r]
            b_slots[SLOT_OF_REL[r]] = zb
        for d in range(3):
            w_slots[ROOT_SLOT[d]] = sum(W_root[l, r] for r in DST_RELS[d])
            b_slots[ROOT_SLOT[d]] = sum(b_rel[l, r] for r in DST_RELS[d])[None, :]
        return jnp.stack(w_slots), jnp.stack(b_slots)

    x3 = jnp.stack([x_a0, x_a1, x_b])
    ws0, bs0 = layer_weights(0)
    t0 = _xform(x3, ws0, bs0)
    accA0 = _sc_scatter((0, 1), (t0,), src_idx, dst_idx, zeros)
    accB0 = _sc_scatter((2,), (t0,), src_idx, dst_idx, zeros)

    ws1, bs1 = layer_weights(1)
    h01 = _combine(accA0, t0, lambda t: 4 * t + 3)
    t1a = _xform(h01, ws1[:8], bs1[:8])
    h2 = _combine(accB0, t0, lambda t: 9)
    t1b = _xform(h2, ws1[8:], bs1[8:], src_div=2)

    accA1 = _sc_scatter((0, 1), (t1a,), src_idx, dst_idx, zeros)
    accB1 = _sc_scatter((2,), (t1a, t1b), src_idx, dst_idx, zeros,
                        slot_map={1: (0, 1), 5: (0, 5), 8: (1, 0)})

    batch3 = jnp.stack([batch_a0, batch_a1, batch_b])[:, None, :]
    psA, csA = _pool(batch3[:2], accA1, t1a, lambda t: 4 * t + 3)
    psB, csB = _pool(batch3[2:], accB1, t1b, lambda t: 1)
    ps = jnp.concatenate([psA, psB])
    cs = jnp.concatenate([csA, csB])

    wo_pad = jnp.pad(Wout, ((0, 0), (0, HD - 1)))
    bo_pad = jnp.pad(bout[None, :], ((0, 0), (0, HD - 1)))
    out = _mlp(ps, cs, W1, b1[None, :], W2, b2[None, :], W3, b3[None, :],
               wo_pad, bo_pad)
    return out[:, 0]


# speedup vs baseline: 1.0702x; 1.0505x over previous
import functools

import jax
import jax.numpy as jnp
from jax import lax
from jax.experimental import pallas as pl
from jax.experimental.pallas import tpu as pltpu
from jax.experimental.pallas import tpu_sc as plsc

N = 10000
E = 320000
F = 128
HD = 128
G = 64
HL = 3 * HD

REL_SRC = (0, 1, 0, 1, 2, 0, 1)
REL_DST = (1, 0, 2, 2, 2, 0, 1)
DST_RELS = ((1, 5), (0, 6), (2, 3, 4))

SLOT_OF_REL = (0, 4, 1, 5, 8, 2, 6)
ROOT_SLOT = (3, 7, 9)
DST_PAIRS = tuple(tuple((SLOT_OF_REL[r], r) for r in rels)
                  for rels in DST_RELS)

NCORE = 2
NSUB = 16
NW = NCORE * NSUB
EW = E // NW
CH = 100
NHALF = 2
NH = EW // (CH * NHALF)
RPS = 624
TAIL = N - NSUB * RPS

BLK = 1000
NB = N // BLK


def _xform_body(x_ref, w_ref, b_ref, o_ref):
    o_ref[...] = (jnp.dot(x_ref[0], w_ref[0], preferred_element_type=jnp.float32)
                  + b_ref[0])[None]


def _xform(x3, ws, bs, src_div=4):
    nout = ws.shape[0]
    return pl.pallas_call(
        _xform_body,
        grid=(NB, nout),
        in_specs=[
            pl.BlockSpec((1, BLK, F), lambda i, o: (o // src_div, i, 0)),
            pl.BlockSpec((1, F, HD), lambda i, o: (o, 0, 0)),
            pl.BlockSpec((1, 1, HD), lambda i, o: (o, 0, 0)),
        ],
        out_specs=pl.BlockSpec((1, BLK, HD), lambda i, o: (o, i, 0)),
        out_shape=jax.ShapeDtypeStruct((nout, N, HD), jnp.float32),
    )(x3, ws, bs)


def _combine_body(a_ref, r_ref, o_ref):
    o_ref[...] = jnp.maximum(a_ref[0, 0] + a_ref[0, 1] + r_ref[0], 0.0)[None]


def _combine(acc, t_planes, root_fn):
    k = acc.shape[0]
    return pl.pallas_call(
        _combine_body,
        grid=(k, NB),
        in_specs=[
            pl.BlockSpec((1, NCORE, BLK, HD), lambda t, i: (t, 0, i, 0)),
            pl.BlockSpec((1, BLK, HD), lambda t, i: (root_fn(t), i, 0)),
        ],
        out_specs=pl.BlockSpec((1, BLK, HD), lambda t, i: (t, i, 0)),
        out_shape=jax.ShapeDtypeStruct((k, N, HD), jnp.float32),
    )(acc, t_planes)


def _pool_body(b_ref, a_ref, r_ref, ps_ref, cs_ref):
    h = jnp.maximum(a_ref[0, 0] + a_ref[0, 1] + r_ref[0], 0.0)
    seg = lax.broadcasted_iota(jnp.int32, (G, N), 0)
    onehot = (jnp.broadcast_to(b_ref[0], (G, N)) == seg).astype(jnp.float32)
    ps_ref[0] = jnp.dot(onehot, h, preferred_element_type=jnp.float32)
    cs_ref[0] = jnp.broadcast_to(jnp.sum(onehot, axis=1, keepdims=True), (G, HD))


def _pool(batchk, acc, t_planes, root_fn):
    k = acc.shape[0]
    return pl.pallas_call(
        _pool_body,
        grid=(k,),
        in_specs=[
            pl.BlockSpec((1, 1, N), lambda t: (t, 0, 0)),
            pl.BlockSpec((1, NCORE, N, HD), lambda t: (t, 0, 0, 0)),
            pl.BlockSpec((1, N, HD), lambda t: (root_fn(t), 0, 0)),
        ],
        out_specs=[
            pl.BlockSpec((1, G, HD), lambda t: (t, 0, 0)),
            pl.BlockSpec((1, G, HD), lambda t: (t, 0, 0)),
        ],
        out_shape=[
            jax.ShapeDtypeStruct((k, G, HD), jnp.float32),
            jax.ShapeDtypeStruct((k, G, HD), jnp.float32),
        ],
    )(batchk, acc, t_planes)


def _mlp_body(ps_ref, cs_ref, w1_ref, b1_ref, w2_ref, b2_ref, w3_ref, b3_ref,
              wo_ref, bo_ref, o_ref):
    pool = ps_ref[...] / jnp.maximum(cs_ref[...], 1.0)
    h = jnp.concatenate([pool[0], pool[1], pool[2]], axis=1)
    h = jnp.maximum(jnp.dot(h, w1_ref[...], preferred_element_type=jnp.float32)
                    + b1_ref[...], 0.0)
    h = jnp.maximum(jnp.dot(h, w2_ref[...], preferred_element_type=jnp.float32)
                    + b2_ref[...], 0.0)
    h = jnp.maximum(jnp.dot(h, w3_ref[...], preferred_element_type=jnp.float32)
                    + b3_ref[...], 0.0)
    o_ref[...] = jnp.dot(h, wo_ref[...], preferred_element_type=jnp.float32) + bo_ref[...]


def _mlp(ps, cs, w1, b1, w2, b2, w3, b3, wo_pad, bo_pad):
    return pl.pallas_call(
        _mlp_body,
        out_shape=jax.ShapeDtypeStruct((G, HD), jnp.float32),
    )(ps, cs, w1, b1, w2, b2, w3, b3, wo_pad, bo_pad)


def _sc_scatter_body(groups, slot_map, n_y, *refs):
    ys = refs[:n_y]
    edges = refs[n_y:n_y + 7]
    (z_hbm, out_hbm,
     src_v, dst_v, rows0, rows1, acc, sem0, sem1) = refs[n_y + 7:]
    c = lax.axis_index("c")
    s = lax.axis_index("s")
    wid = s * NCORE + c
    row0 = s * RPS
    for gi, d in enumerate(groups):
        pltpu.sync_copy(z_hbm.at[pl.ds(0, RPS)], acc.at[pl.ds(row0, RPS)])

        @pl.when(s == NSUB - 1)
        def _():
            pltpu.sync_copy(z_hbm.at[pl.ds(0, TAIL)],
                            acc.at[pl.ds(N - TAIL, TAIL)])

        plsc.subcore_barrier()
        for slot, r in DST_PAIRS[d]:
            ai, plane = slot_map[slot]
            y_slot = ys[ai].at[plane]
            for h in range(NHALF):
                pltpu.sync_copy(edges[r].at[0, wid, h], src_v)
                pltpu.sync_copy(edges[r].at[1, wid, h], dst_v)

                pltpu.async_copy(y_slot.at[src_v.at[0]], rows0, sem0)

                def body(p, carry, y_slot=y_slot):
                    c0, c1, c2 = 2 * p, 2 * p + 1, 2 * p + 2
                    pltpu.async_copy(y_slot.at[src_v.at[c1]], rows1, sem1)
                    pltpu.make_async_copy(y_slot.at[src_v.at[c0]], rows0, sem0).wait()
                    pltpu.sync_copy(rows0, acc.at[dst_v.at[c0]], add=True)
                    pltpu.async_copy(y_slot.at[src_v.at[c2]], rows0, sem0)
                    pltpu.make_async_copy(y_slot.at[src_v.at[c1]], rows1, sem1).wait()
                    pltpu.sync_copy(rows1, acc.at[dst_v.at[c1]], add=True)
                    return carry

                lax.fori_loop(0, NH // 2 - 1, body, 0)
                pltpu.async_copy(y_slot.at[src_v.at[NH - 1]], rows1, sem1)
                pltpu.make_async_copy(y_slot.at[src_v.at[NH - 2]], rows0, sem0).wait()
                pltpu.sync_copy(rows0, acc.at[dst_v.at[NH - 2]], add=True)
                pltpu.make_async_copy(y_slot.at[src_v.at[NH - 1]], rows1, sem1).wait()
                pltpu.sync_copy(rows1, acc.at[dst_v.at[NH - 1]], add=True)
        plsc.subcore_barrier()
        pltpu.sync_copy(acc.at[pl.ds(row0, RPS)],
                        out_hbm.at[gi, c, pl.ds(row0, RPS)])

        @pl.when(s == NSUB - 1)
        def _():
            pltpu.sync_copy(acc.at[pl.ds(N - TAIL, TAIL)],
                            out_hbm.at[gi, c, pl.ds(N - TAIL, TAIL)])

        plsc.subcore_barrier()


@functools.cache
def _sc_scatter_kernel(groups, slot_map, n_y):
    return pl.kernel(
        functools.partial(_sc_scatter_body, groups, dict(slot_map), n_y),
        out_type=jax.ShapeDtypeStruct((len(groups), NCORE, N, HD), jnp.float32),
        mesh=plsc.VectorSubcoreMesh(core_axis_name="c", subcore_axis_name="s",
                                    num_cores=NCORE, num_subcores=NSUB),
        scratch_types=[
            pltpu.VMEM((NH, CH), jnp.int32),
            pltpu.VMEM((NH, CH), jnp.int32),
            pltpu.VMEM((CH, HD), jnp.float32),
            pltpu.VMEM((CH, HD), jnp.float32),
            pltpu.VMEM_SHARED((N, HD), jnp.float32),
            pltpu.SemaphoreType.DMA,
            pltpu.SemaphoreType.DMA,
        ],
    )


def _sc_scatter(groups, ys, edges, zeros, slot_map=None):
    if slot_map is None:
        slot_map = {slot: (0, slot) for d in groups for slot, _ in DST_PAIRS[d]}
    k = _sc_scatter_kernel(tuple(groups), tuple(sorted(slot_map.items())), len(ys))
    return k(*ys, *edges, zeros)


def kernel(x_a0, x_a1, x_b, ei_a0_a1, ei_a1_a0, ei_a0_b, ei_a1_b, ei_b_b,
           ei_a0_a0, ei_a1_a1, batch_a0, batch_a1, batch_b, W_rel, b_rel,
           W_root, W1, b1, W2, b2, W3, b3, Wout, bout):
    eis = (ei_a0_a1, ei_a1_a0, ei_a0_b, ei_a1_b, ei_b_b, ei_a0_a0, ei_a1_a1)
    edges = tuple(e.reshape(2, NW, NHALF, NH, CH) for e in eis)
    zeros = jnp.zeros((RPS, HD), jnp.float32)

    def layer_weights(l):
        zb = jnp.zeros((7, 1, HD), jnp.float32)
        w_slots, b_slots = [None] * 10, [None] * 10
        for r in range(7):
            w_slots[SLOT_OF_REL[r]] = W_rel[l, r:r + 1]
            b_slots[SLOT_OF_REL[r]] = zb[r:r + 1]
        for d in range(3):
            w_slots[ROOT_SLOT[d]] = sum(W_root[l, r] for r in DST_RELS[d])[None]
            b_slots[ROOT_SLOT[d]] = sum(b_rel[l, r] for r in DST_RELS[d])[None, None, :]
        return jnp.concatenate(w_slots), jnp.concatenate(b_slots)

    x3 = jnp.stack([x_a0, x_a1, x_b])
    ws0, bs0 = layer_weights(0)
    t0 = _xform(x3, ws0, bs0)
    accA0 = _sc_scatter((0, 1), (t0,), edges, zeros)
    zeros_b0 = zeros + 0.0 * accA0[0, 0, 0, 0]
    accB0 = _sc_scatter((2,), (t0,), edges, zeros_b0)

    ws1, bs1 = layer_weights(1)
    h01 = _combine(accA0, t0, lambda t: 4 * t + 3)
    t1a = _xform(h01, ws1[:8], bs1[:8])
    h2 = _combine(accB0, t0, lambda t: 9)
    t1b = _xform(h2, ws1[8:], bs1[8:], src_div=2)

    accA1 = _sc_scatter((0, 1), (t1a,), edges, zeros)
    zeros_b1 = zeros + 0.0 * accA1[0, 0, 0, 0]
    accB1 = _sc_scatter((2,), (t1a, t1b), edges, zeros_b1,
                        slot_map={1: (0, 1), 5: (0, 5), 8: (1, 0)})

    batch3 = jnp.stack([batch_a0, batch_a1, batch_b])[:, None, :]
    psA, csA = _pool(batch3[:2], accA1, t1a, lambda t: 4 * t + 3)
    psB, csB = _pool(batch3[2:], accB1, t1b, lambda t: 1)
    ps = jnp.concatenate([psA, psB])
    cs = jnp.concatenate([csA, csB])

    wo_pad = jnp.pad(Wout, ((0, 0), (0, HD - 1)))
    bo_pad = jnp.pad(bout[None, :], ((0, 0), (0, HD - 1)))
    out = _mlp(ps, cs, W1, b1[None, :], W2, b2[None, :], W3, b3[None, :],
               wo_pad, bo_pad)
    return out[:, 0]
